# baseline (speedup 1.0000x reference)
import jax
import jax.numpy as jnp
from jax import lax
from jax.experimental import pallas as pl
from jax.experimental.pallas import tpu as pltpu
from jax.experimental.pallas import tpu_sc as plsc

N_NEWS = 8192
V = 100000
D = 32
C = 16
L = 50
H = 4
DH = D // H
OC = 32
TW = 128

NC = 2
NS = 16
NW = NC * NS
VP = 100352
WPW = VP // NW
BW = 16
NG = WPW // BW
NDOC = N_NEWS * L
LP = 52
NDOCP = N_NEWS * LP
DOC_CH = N_NEWS // (NW * 2)

_SC_PARAMS = pltpu.CompilerParams(
    needs_layout_passes=False, disable_bounds_checks=True)


def _proj_tc_kernel(ne_ref, w_ref, b_ref, t_ref):
    t_ref[...] = (
        jnp.dot(ne_ref[...], w_ref[...], preferred_element_type=jnp.float32)
        + b_ref[...]
    )


def _attn_compute(rows, outb, sbuf, bqv, iota, oiv):
    riv = [iota * C + i for i in range(C)]
    zv = jnp.zeros((BW,), jnp.int32)

    def head_body(h, carry):
        cq = h * DH
        ck = D + cq
        cv = 2 * D + cq
        cols_q = [zv + (cq + j) for j in range(DH)]
        cols_k = [zv + (ck + j) for j in range(DH)]
        cols_v = [zv + (cv + j) for j in range(DH)]

        qacc = [plsc.load_gather(rows, [riv[0], cols_q[j]]) for j in range(DH)]
        for i in range(1, C):
            for j in range(DH):
                qacc[j] = qacc[j] + plsc.load_gather(rows, [riv[i], cols_q[j]])
        q = [qacc[j] + plsc.load_gather(bqv, [cols_q[j]]) for j in range(DH)]

        for i in range(C):
            si = q[0] * plsc.load_gather(rows, [riv[i], cols_k[0]])
            for j in range(1, DH):
                si = si + q[j] * plsc.load_gather(rows, [riv[i], cols_k[j]])
            sbuf[pl.ds(i * BW, BW)] = si

        s = [sbuf[pl.ds(i * BW, BW)] for i in range(C)]
        m = s[0]
        for i in range(1, C):
            m = jnp.maximum(m, s[i])
        e = [jnp.exp(si - m) for si in s]
        z = e[0]
        for i in range(1, C):
            z = z + e[i]
        rz = 1.0 / z
        for i in range(C):
            sbuf[pl.ds(i * BW, BW)] = e[i]

        oacc = [None] * DH
        for i in range(C):
            ei = sbuf[pl.ds(i * BW, BW)]
            for j in range(DH):
                t = ei * plsc.load_gather(rows, [riv[i], cols_v[j]])
                oacc[j] = t if i == 0 else oacc[j] + t
        for j in range(DH):
            plsc.store_scatter(outb, [oiv + (cq + j)], oacc[j] * rz)
        return carry

    lax.fori_loop(0, H, head_body, 0, unroll=2)


def _attn_sc_body(iota_ref, ids_ref, bq_ref, t_ref, o_ref,
                  ib0, ib1, rb0, rb1, ob0, ob1, bqv, sbuf, itv,
                  sg0, sg1, so0, so1, si0, si1):
    wid = lax.axis_index("s") * NC + lax.axis_index("c")
    pltpu.sync_copy(bq_ref, bqv)
    pltpu.sync_copy(iota_ref, itv)
    iota = itv[pl.ds(0, BW)]
    oiv = iota * TW
    ibs, rbs, obs = (ib0, ib1), (rb0, rb1), (ob0, ob1)
    sgs, sos, sis = (sg0, sg1), (so0, so1), (si0, si1)
    irow0 = wid * (WPW * C // 128)

    def fire(p):
        pltpu.async_copy(t_ref.at[ibs[p].at[0]],
                         rbs[p].at[pl.ds(0, 128)], sgs[p])
        pltpu.async_copy(t_ref.at[ibs[p].at[1]],
                         rbs[p].at[pl.ds(128, 128)], sgs[p])

    pltpu.sync_copy(ids_ref.at[pl.ds(irow0, 2)], ib0)
    fire(0)
    pltpu.async_copy(ids_ref.at[pl.ds(irow0 + 2, 2)], ib1, si1)

    def step(it, carry):
        for p in range(2):
            g = it * 2 + p

            @pl.when(g + 1 < NG)
            def _():
                pltpu.make_async_copy(ids_ref.at[pl.ds(0, 2)],
                                      ibs[1 - p], sis[1 - p]).wait()
                fire(1 - p)

            pltpu.make_async_copy(t_ref.at[pl.ds(0, BW * C)],
                                  rbs[p], sgs[p]).wait()

            @pl.when(g + 2 < NG)
            def _():
                pltpu.async_copy(ids_ref.at[pl.ds(irow0 + (g + 2) * 2, 2)],
                                 ibs[p], sis[p])

            @pl.when(g >= 2)
            def _():
                pltpu.make_async_copy(o_ref.at[pl.ds(0, BW * TW)],
                                      obs[p], sos[p]).wait()

            _attn_compute(rbs[p], obs[p], sbuf, bqv, iota, oiv)
            base = wid * WPW + g * BW
            pltpu.async_copy(obs[p], o_ref.at[pl.ds(base * TW, BW * TW)],
                             sos[p])
        return carry

    lax.fori_loop(0, NG // 2, step, 0)
    for p in range(2):
        pltpu.make_async_copy(o_ref.at[pl.ds(0, BW * TW)],
                              obs[p], sos[p]).wait()


def _doc_sc_body(nwi_ref, o_ref, doc_ref, idxb, rowsb, docb, sem):
    wid = lax.axis_index("s") * NC + lax.axis_index("c")

    def step(c, carry):
        r = wid * DOC_CH + c
        pltpu.sync_copy(nwi_ref.at[r], idxb)
        pltpu.async_copy(o_ref.at[idxb], rowsb, sem).wait()
        for half in range(2):
            ob = half * 52 * D
            ib = half * 50
            for p in range(50):
                docb[pl.ds(ob + p * D, 16)] = rowsb[ib + p, pl.ds(0, 16)]
                docb[pl.ds(ob + p * D + 16, 16)] = rowsb[ib + p, pl.ds(16, 16)]
        pltpu.sync_copy(docb, doc_ref.at[pl.ds(r * 104 * D, 104 * D)])
        return carry

    lax.fori_loop(0, DOC_CH, step, 0)


def _cnn_tc_kernel(doc_ref, wo_ref, bo_ref, wc_ref, b3_ref, b5_ref,
                   fw_ref, fb_ref, out_ref):
    gn = out_ref.shape[0]
    rows = gn * LP * D // 128
    x = jnp.dot(doc_ref[...], wo_ref[...],
                preferred_element_type=jnp.float32) + bo_ref[...]
    z = jnp.dot(x, wc_ref[...], preferred_element_type=jnp.float32)
    z1 = jnp.concatenate([z[1:], z[:1]], axis=0)

    def term(j, k, off):
        jb = j + k
        srcz = z if jb < 4 else z1
        jb = jb % 4
        return srcz[:, jb * 256 + off + 32 * k: jb * 256 + off + 32 * k + 32]

    y3 = jnp.concatenate(
        [term(j, 0, 0) + term(j, 1, 0) + term(j, 2, 0) for j in range(4)],
        axis=1) + b3_ref[...]
    y5 = jnp.concatenate(
        [term(j, 0, 96) + term(j, 1, 96) + term(j, 2, 96)
         + term(j, 3, 96) + term(j, 4, 96) for j in range(4)],
        axis=1) + b5_ref[...]
    pos = (lax.broadcasted_iota(jnp.int32, (rows, 128), 0) * 4
           + lax.broadcasted_iota(jnp.int32, (rows, 128), 1) // 32) % LP
    y3 = jnp.where(pos < L - 2, jax.nn.relu(y3), 0.0)
    y5 = jnp.where(pos < L - 4, jax.nn.relu(y5), 0.0)
    m3 = jnp.max(y3.reshape(gn, LP * D // 128, 128), axis=1)
    m5 = jnp.max(y5.reshape(gn, LP * D // 128, 128), axis=1)
    p3 = jnp.maximum(jnp.maximum(m3[:, 0:32], m3[:, 32:64]),
                     jnp.maximum(m3[:, 64:96], m3[:, 96:128]))
    p5 = jnp.maximum(jnp.maximum(m5[:, 0:32], m5[:, 32:64]),
                     jnp.maximum(m5[:, 64:96], m5[:, 96:128]))
    feat = jnp.concatenate([p3, p5], axis=1)
    out_ref[...] = (
        jnp.dot(feat, fw_ref[...], preferred_element_type=jnp.float32)
        + fb_ref[...]
    )


def kernel(all_news_ids, word_news_ids, news_word_ids, news_embeds,
           Wq, bq, Wk, bk, Wv, bv, Wo, bo,
           conv_w3, conv_b3, conv_w5, conv_b5, fc_w, fc_b):
    f32 = jnp.float32
    scale = 1.0 / jnp.sqrt(jnp.array(DH, f32))
    wcat = jnp.concatenate(
        [Wq.T / C, Wk.T * scale, Wv.T, jnp.zeros((D, TW - 3 * D), f32)], axis=1)
    bcat = jnp.concatenate(
        [jnp.zeros((D,), f32), bk * scale, bv, jnp.zeros((TW - 3 * D,), f32)]
    )[None, :]
    table = pl.pallas_call(
        _proj_tc_kernel,
        out_shape=jax.ShapeDtypeStruct((N_NEWS, TW), f32),
    )(news_embeds, wcat, bcat)

    ids_pad = jnp.pad(word_news_ids, ((0, VP - V), (0, 0)))
    ids2d = ids_pad.reshape(VP * C // 128, 128)
    mesh = plsc.VectorSubcoreMesh(core_axis_name="c", subcore_axis_name="s")
    attn = pl.kernel(
        _attn_sc_body,
        out_type=jax.ShapeDtypeStruct((VP * TW,), f32),
        mesh=mesh,
        compiler_params=_SC_PARAMS,
        scratch_types=[
            pltpu.VMEM((2, 128), jnp.int32),
            pltpu.VMEM((2, 128), jnp.int32),
            pltpu.VMEM((BW * C, TW), f32),
            pltpu.VMEM((BW * C, TW), f32),
            pltpu.VMEM((BW * TW,), f32),
            pltpu.VMEM((BW * TW,), f32),
            pltpu.VMEM((D,), f32),
            pltpu.VMEM((C * BW,), f32),
            pltpu.VMEM((128,), jnp.int32),
            pltpu.SemaphoreType.DMA,
            pltpu.SemaphoreType.DMA,
            pltpu.SemaphoreType.DMA,
            pltpu.SemaphoreType.DMA,
            pltpu.SemaphoreType.DMA,
            pltpu.SemaphoreType.DMA,
        ],
    )
    iota128 = jnp.arange(128, dtype=jnp.int32)
    o_tab = attn(iota128, ids2d, bq, table).reshape(VP, TW)

    nwi2d = jnp.pad(news_word_ids.reshape(N_NEWS // 2, 100), ((0, 0), (0, 4)))
    docg = pl.kernel(
        _doc_sc_body,
        out_type=jax.ShapeDtypeStruct((NDOCP * D,), f32),
        mesh=mesh,
        compiler_params=_SC_PARAMS,
        scratch_types=[
            pltpu.VMEM((104,), jnp.int32),
            pltpu.VMEM((104, TW), f32),
            pltpu.VMEM((104 * D,), f32),
            pltpu.SemaphoreType.DMA,
        ],
    )
    doc = docg(nwi2d, o_tab).reshape(NDOCP * D // 128, 128)

    wc2 = jnp.concatenate(
        [conv_w3[:, :, j].T for j in range(3)]
        + [conv_w5[:, :, j].T for j in range(5)], axis=1)

    def bdiag(w, n):
        r, c = w.shape
        out = jnp.zeros((n * r, n * c), f32)
        for j in range(n):
            out = out.at[j * r:(j + 1) * r, j * c:(j + 1) * c].set(w)
        return out

    wo_bd = bdiag(Wo.T, 4)
    wc2_bd = bdiag(wc2, 4)
    bo_p = jnp.tile(bo, 4)[None, :]
    b3_p = jnp.tile(conv_b3, 4)[None, :]
    b5_p = jnp.tile(conv_b5, 4)[None, :]
    GN = 64
    grid = N_NEWS // GN
    brows = GN * LP * D // 128
    out = pl.pallas_call(
        _cnn_tc_kernel,
        grid=(grid,),
        in_specs=[
            pl.BlockSpec((brows, 128), lambda i: (i, 0)),
            pl.BlockSpec((128, 128), lambda i: (0, 0)),
            pl.BlockSpec((1, 128), lambda i: (0, 0)),
            pl.BlockSpec((128, 8 * OC * 4), lambda i: (0, 0)),
            pl.BlockSpec((1, 128), lambda i: (0, 0)),
            pl.BlockSpec((1, 128), lambda i: (0, 0)),
            pl.BlockSpec((2 * OC, D), lambda i: (0, 0)),
            pl.BlockSpec((1, D), lambda i: (0, 0)),
        ],
        out_specs=pl.BlockSpec((GN, D), lambda i: (i, 0)),
        out_shape=jax.ShapeDtypeStruct((N_NEWS, D), f32),
    )(doc, wo_bd, bo_p, wc2_bd, b3_p, b5_p, fc_w.T, fc_b[None, :])
    return out


# speedup vs baseline: 1.1188x; 1.1188x over previous
import jax
import jax.numpy as jnp
from jax import lax
from jax.experimental import pallas as pl
from jax.experimental.pallas import tpu as pltpu
from jax.experimental.pallas import tpu_sc as plsc

N_NEWS = 8192
V = 100000
D = 32
C = 16
L = 50
H = 4
DH = D // H
OC = 32
TW = 128

NC = 2
NS = 16
NW = NC * NS
VP = 100352
WPW = VP // NW
BW = 16
NG = WPW // BW
NDOC = N_NEWS * L
DOC_CH = NDOC // (NW * 128)

_SC_PARAMS = pltpu.CompilerParams(
    needs_layout_passes=False, disable_bounds_checks=True)


def _proj_tc_kernel(ne_ref, w_ref, b_ref, t_ref):
    t_ref[...] = (
        jnp.dot(ne_ref[...], w_ref[...], preferred_element_type=jnp.float32)
        + b_ref[...]
    )


def _attn_compute(rows, outb, sbuf, bqv, iota, oiv):
    riv = [iota * C + i for i in range(C)]
    zv = jnp.zeros((BW,), jnp.int32)

    def head_body(h, carry):
        cq = h * DH
        ck = D + cq
        cv = 2 * D + cq
        cols_q = [zv + (cq + j) for j in range(DH)]
        cols_k = [zv + (ck + j) for j in range(DH)]
        cols_v = [zv + (cv + j) for j in range(DH)]

        qacc = [plsc.load_gather(rows, [riv[0], cols_q[j]]) for j in range(DH)]
        for i in range(1, C):
            for j in range(DH):
                qacc[j] = qacc[j] + plsc.load_gather(rows, [riv[i], cols_q[j]])
        q = [qacc[j] + plsc.load_gather(bqv, [cols_q[j]]) for j in range(DH)]

        for i in range(C):
            si = q[0] * plsc.load_gather(rows, [riv[i], cols_k[0]])
            for j in range(1, DH):
                si = si + q[j] * plsc.load_gather(rows, [riv[i], cols_k[j]])
            sbuf[pl.ds(i * BW, BW)] = si

        s = [sbuf[pl.ds(i * BW, BW)] for i in range(C)]
        m = s[0]
        for i in range(1, C):
            m = jnp.maximum(m, s[i])
        e = [jnp.exp(si - m) for si in s]
        z = e[0]
        for i in range(1, C):
            z = z + e[i]
        rz = 1.0 / z
        for i in range(C):
            sbuf[pl.ds(i * BW, BW)] = e[i]

        oacc = [None] * DH
        for i in range(C):
            ei = sbuf[pl.ds(i * BW, BW)]
            for j in range(DH):
                t = ei * plsc.load_gather(rows, [riv[i], cols_v[j]])
                oacc[j] = t if i == 0 else oacc[j] + t
        for j in range(DH):
            plsc.store_scatter(outb, [oiv + (cq + j)], oacc[j] * rz)
        return carry

    lax.fori_loop(0, H, head_body, 0, unroll=2)


def _attn_sc_body(iota_ref, ids_ref, bq_ref, t_ref, o_ref,
                  ib0, ib1, rb0, rb1, ob0, ob1, bqv, sbuf, itv,
                  sg0, sg1, so0, so1, si0, si1):
    wid = lax.axis_index("s") * NC + lax.axis_index("c")
    pltpu.sync_copy(bq_ref, bqv)
    pltpu.sync_copy(iota_ref, itv)
    iota = itv[pl.ds(0, BW)]
    oiv = iota * TW
    ibs, rbs, obs = (ib0, ib1), (rb0, rb1), (ob0, ob1)
    sgs, sos, sis = (sg0, sg1), (so0, so1), (si0, si1)
    irow0 = wid * (WPW * C // 128)

    def fire(p):
        pltpu.async_copy(t_ref.at[ibs[p].at[0]],
                         rbs[p].at[pl.ds(0, 128)], sgs[p])
        pltpu.async_copy(t_ref.at[ibs[p].at[1]],
                         rbs[p].at[pl.ds(128, 128)], sgs[p])

    pltpu.sync_copy(ids_ref.at[pl.ds(irow0, 2)], ib0)
    fire(0)
    pltpu.async_copy(ids_ref.at[pl.ds(irow0 + 2, 2)], ib1, si1)

    def step(it, carry):
        for p in range(2):
            g = it * 2 + p

            @pl.when(g + 1 < NG)
            def _():
                pltpu.make_async_copy(ids_ref.at[pl.ds(0, 2)],
                                      ibs[1 - p], sis[1 - p]).wait()
                fire(1 - p)

            pltpu.make_async_copy(t_ref.at[pl.ds(0, BW * C)],
                                  rbs[p], sgs[p]).wait()

            @pl.when(g + 2 < NG)
            def _():
                pltpu.async_copy(ids_ref.at[pl.ds(irow0 + (g + 2) * 2, 2)],
                                 ibs[p], sis[p])

            @pl.when(g >= 2)
            def _():
                pltpu.make_async_copy(o_ref.at[pl.ds(0, BW * TW)],
                                      obs[p], sos[p]).wait()

            _attn_compute(rbs[p], obs[p], sbuf, bqv, iota, oiv)
            base = wid * WPW + g * BW
            pltpu.async_copy(obs[p], o_ref.at[pl.ds(base * TW, BW * TW)],
                             sos[p])
        return carry

    lax.fori_loop(0, NG // 2, step, 0)
    for p in range(2):
        pltpu.make_async_copy(o_ref.at[pl.ds(0, BW * TW)],
                              obs[p], sos[p]).wait()


def _doc_sc_body(nwi_ref, o_ref, doc_ref, idxb, rowsb, docb, sem):
    wid = lax.axis_index("s") * NC + lax.axis_index("c")

    def step(c, carry):
        r = wid * DOC_CH + c
        pltpu.sync_copy(nwi_ref.at[r], idxb)
        pltpu.async_copy(o_ref.at[idxb], rowsb, sem).wait()
        for p in range(128):
            docb[pl.ds(p * D, 16)] = rowsb[p, pl.ds(0, 16)]
            docb[pl.ds(p * D + 16, 16)] = rowsb[p, pl.ds(16, 16)]
        pltpu.sync_copy(docb, doc_ref.at[pl.ds(r * 128 * D, 128 * D)])
        return carry

    lax.fori_loop(0, DOC_CH, step, 0)


def _cnn_tc_kernel(doc_ref, wo_ref, bo_ref, wc_ref, b3_ref, b5_ref,
                   fw_ref, fb_ref, out_ref):
    gn = out_ref.shape[0]
    rows = gn * L
    x = jnp.dot(doc_ref[...], wo_ref[...],
                preferred_element_type=jnp.float32) + bo_ref[...]
    z = jnp.dot(x, wc_ref[...], preferred_element_type=jnp.float32)

    def rot(a, j):
        return a if j == 0 else jnp.concatenate([a[j:], a[:j]], axis=0)

    t = lax.broadcasted_iota(jnp.int32, (rows, 1), 0) % L
    y3 = (z[:, 0:32] + rot(z[:, 32:64], 1) + rot(z[:, 64:96], 2)
          + b3_ref[...])
    y3 = jnp.where(t < L - 2, jax.nn.relu(y3), 0.0)
    y5 = (z[:, 96:128] + rot(z[:, 128:160], 1) + rot(z[:, 160:192], 2)
          + rot(z[:, 192:224], 3) + rot(z[:, 224:256], 4) + b5_ref[...])
    y5 = jnp.where(t < L - 4, jax.nn.relu(y5), 0.0)
    p3 = jnp.max(y3.reshape(gn, L, OC), axis=1)
    p5 = jnp.max(y5.reshape(gn, L, OC), axis=1)
    feat = jnp.concatenate([p3, p5], axis=1)
    out_ref[...] = (
        jnp.dot(feat, fw_ref[...], preferred_element_type=jnp.float32)
        + fb_ref[...]
    )


def kernel(all_news_ids, word_news_ids, news_word_ids, news_embeds,
           Wq, bq, Wk, bk, Wv, bv, Wo, bo,
           conv_w3, conv_b3, conv_w5, conv_b5, fc_w, fc_b):
    f32 = jnp.float32
    scale = 1.0 / jnp.sqrt(jnp.array(DH, f32))
    wcat = jnp.concatenate(
        [Wq.T / C, Wk.T * scale, Wv.T, jnp.zeros((D, TW - 3 * D), f32)], axis=1)
    bcat = jnp.concatenate(
        [jnp.zeros((D,), f32), bk * scale, bv, jnp.zeros((TW - 3 * D,), f32)]
    )[None, :]
    table = pl.pallas_call(
        _proj_tc_kernel,
        out_shape=jax.ShapeDtypeStruct((N_NEWS, TW), f32),
    )(news_embeds, wcat, bcat)

    ids_pad = jnp.pad(word_news_ids, ((0, VP - V), (0, 0)))
    ids2d = ids_pad.reshape(VP * C // 128, 128)
    mesh = plsc.VectorSubcoreMesh(core_axis_name="c", subcore_axis_name="s")
    attn = pl.kernel(
        _attn_sc_body,
        out_type=jax.ShapeDtypeStruct((VP * TW,), f32),
        mesh=mesh,
        compiler_params=_SC_PARAMS,
        scratch_types=[
            pltpu.VMEM((2, 128), jnp.int32),
            pltpu.VMEM((2, 128), jnp.int32),
            pltpu.VMEM((BW * C, TW), f32),
            pltpu.VMEM((BW * C, TW), f32),
            pltpu.VMEM((BW * TW,), f32),
            pltpu.VMEM((BW * TW,), f32),
            pltpu.VMEM((D,), f32),
            pltpu.VMEM((C * BW,), f32),
            pltpu.VMEM((128,), jnp.int32),
            pltpu.SemaphoreType.DMA,
            pltpu.SemaphoreType.DMA,
            pltpu.SemaphoreType.DMA,
            pltpu.SemaphoreType.DMA,
            pltpu.SemaphoreType.DMA,
            pltpu.SemaphoreType.DMA,
        ],
    )
    iota128 = jnp.arange(128, dtype=jnp.int32)
    o_tab = attn(iota128, ids2d, bq, table).reshape(VP, TW)

    nwi2d = news_word_ids.reshape(NDOC // 128, 128)
    docg = pl.kernel(
        _doc_sc_body,
        out_type=jax.ShapeDtypeStruct((NDOC * D,), f32),
        mesh=mesh,
        compiler_params=_SC_PARAMS,
        scratch_types=[
            pltpu.VMEM((128,), jnp.int32),
            pltpu.VMEM((128, TW), f32),
            pltpu.VMEM((128 * D,), f32),
            pltpu.SemaphoreType.DMA,
        ],
    )
    doc = docg(nwi2d, o_tab).reshape(NDOC, D)

    wc2 = jnp.concatenate(
        [conv_w3[:, :, j].T for j in range(3)]
        + [conv_w5[:, :, j].T for j in range(5)], axis=1)
    GN = 64
    grid = N_NEWS // GN
    out = pl.pallas_call(
        _cnn_tc_kernel,
        grid=(grid,),
        in_specs=[
            pl.BlockSpec((GN * L, D), lambda i: (i, 0)),
            pl.BlockSpec((D, D), lambda i: (0, 0)),
            pl.BlockSpec((1, D), lambda i: (0, 0)),
            pl.BlockSpec((D, 8 * OC), lambda i: (0, 0)),
            pl.BlockSpec((1, OC), lambda i: (0, 0)),
            pl.BlockSpec((1, OC), lambda i: (0, 0)),
            pl.BlockSpec((2 * OC, D), lambda i: (0, 0)),
            pl.BlockSpec((1, D), lambda i: (0, 0)),
        ],
        out_specs=pl.BlockSpec((GN, D), lambda i: (i, 0)),
        out_shape=jax.ShapeDtypeStruct((N_NEWS, D), f32),
    )(doc, Wo.T, bo[None, :], wc2,
      conv_b3[None, :], conv_b5[None, :], fc_w.T, fc_b[None, :])
    return out
